# Initial kernel scaffold; baseline (speedup 1.0000x reference)
import jax
import jax.numpy as jnp
from jax.experimental import pallas as pl


def kernel(x):
    raise NotImplementedError("write your pallas kernel here")


# speedup vs baseline: 10.3911x; 10.3911x over previous
import jax
import jax.numpy as jnp
from jax.experimental import pallas as pl

_BI = 128


def _strip_kernel(xiT_ref, xjT_ref, out_ref):
    xjT = xjT_ref[:, :]
    for bi in range(_BI):
        col = xiT_ref[:, bi : bi + 1]
        m = jnp.max(jnp.abs(xjT - col), axis=0, keepdims=True)
        out_ref[bi : bi + 1, :] = m


def kernel(x):
    n, d = x.shape
    xT = x.T
    out = pl.pallas_call(
        _strip_kernel,
        grid=(n // _BI,),
        in_specs=[
            pl.BlockSpec((d, _BI), lambda i: (0, i)),
            pl.BlockSpec((d, n), lambda i: (0, 0)),
        ],
        out_specs=pl.BlockSpec((_BI, n), lambda i: (i, 0)),
        out_shape=jax.ShapeDtypeStruct((n, n), x.dtype),
    )(xT, xT)
    return out


# speedup vs baseline: 14.9663x; 1.4403x over previous
import jax
import jax.numpy as jnp
import numpy as np
from jax.experimental import pallas as pl
from jax.experimental.pallas import tpu as pltpu

_T = 256


def _tile_kernel(ij_ref, xiT_ref, xjT_ref, out_u_ref, out_l_ref):
    xjT = xjT_ref[:, :]
    for a in range(_T):
        col = xiT_ref[:, a : a + 1]
        out_u_ref[a : a + 1, :] = jnp.max(
            jnp.abs(xjT - col), axis=0, keepdims=True
        )
    out_l_ref[:, :] = out_u_ref[:, :].T


def _pairwise_inf(xT, ij, n, d, interpret=False):
    nb = n // _T
    npairs = ij.shape[1]
    grid_spec = pltpu.PrefetchScalarGridSpec(
        num_scalar_prefetch=1,
        grid=(npairs,),
        in_specs=[
            pl.BlockSpec((d, _T), lambda t, ij: (0, ij[0, t])),
            pl.BlockSpec((d, _T), lambda t, ij: (0, ij[1, t])),
        ],
        out_specs=[
            pl.BlockSpec((_T, _T), lambda t, ij: (ij[0, t], ij[1, t])),
            pl.BlockSpec((_T, _T), lambda t, ij: (ij[1, t], ij[0, t])),
        ],
    )
    out_u, out_l = pl.pallas_call(
        _tile_kernel,
        grid_spec=grid_spec,
        out_shape=[
            jax.ShapeDtypeStruct((n, n), xT.dtype),
            jax.ShapeDtypeStruct((n, n), xT.dtype),
        ],
        interpret=interpret,
    )(ij, xT, xT)
    r = jax.lax.broadcasted_iota(jnp.int32, (n, n), 0)
    c = jax.lax.broadcasted_iota(jnp.int32, (n, n), 1)
    return jnp.where(c >= r, out_u, out_l)


def kernel(x):
    n, d = x.shape
    nb = n // _T
    pairs = np.array(
        [(i, j) for i in range(nb) for j in range(i, nb)], dtype=np.int32
    ).T
    return _pairwise_inf(x.T, jnp.asarray(pairs), n, d)
